# baseline (device time: 52095 ns/iter reference)
import jax
import jax.numpy as jnp
from jax import lax
from jax.experimental import pallas as pl
from jax.experimental.pallas import tpu as pltpu

N_DEV = 8
B = 2
SQ = 512
SKV = 512
HQ = 8
DH = 64
DM = 768
W = 128
KF = SKV + 2 * W


def kernel(x, Wq, K_ext, V_ext, Wo):
    def body(x_ref, wq_ref, k_ref, v_ref, wo_ref, out_ref,
             kl_ref, kr_ref, vl_ref, vr_ref, send_sems, recv_sems):
        my = lax.axis_index("i")
        has_left = my > 0
        has_right = my < N_DEV - 1
        left = jnp.maximum(my - 1, 0)
        right = jnp.minimum(my + 1, N_DEV - 1)

        kl_ref[...] = jnp.zeros_like(kl_ref)
        kr_ref[...] = jnp.zeros_like(kr_ref)
        vl_ref[...] = jnp.zeros_like(vl_ref)
        vr_ref[...] = jnp.zeros_like(vr_ref)

        barrier = pltpu.get_barrier_semaphore()

        @pl.when(has_left)
        def _():
            pl.semaphore_signal(barrier, inc=1, device_id=(left,),
                                device_id_type=pl.DeviceIdType.MESH)

        @pl.when(has_right)
        def _():
            pl.semaphore_signal(barrier, inc=1, device_id=(right,),
                                device_id_type=pl.DeviceIdType.MESH)

        @pl.when(has_left)
        def _():
            pl.semaphore_wait(barrier, 1)

        @pl.when(has_right)
        def _():
            pl.semaphore_wait(barrier, 1)

        to_right_k = pltpu.make_async_remote_copy(
            src_ref=k_ref.at[:, pl.ds(SKV - W, W)], dst_ref=kl_ref,
            send_sem=send_sems.at[0], recv_sem=recv_sems.at[0],
            device_id=(right,), device_id_type=pl.DeviceIdType.MESH)
        to_right_v = pltpu.make_async_remote_copy(
            src_ref=v_ref.at[:, pl.ds(SKV - W, W)], dst_ref=vl_ref,
            send_sem=send_sems.at[2], recv_sem=recv_sems.at[2],
            device_id=(right,), device_id_type=pl.DeviceIdType.MESH)
        to_left_k = pltpu.make_async_remote_copy(
            src_ref=k_ref.at[:, pl.ds(0, W)], dst_ref=kr_ref,
            send_sem=send_sems.at[1], recv_sem=recv_sems.at[1],
            device_id=(left,), device_id_type=pl.DeviceIdType.MESH)
        to_left_v = pltpu.make_async_remote_copy(
            src_ref=v_ref.at[:, pl.ds(0, W)], dst_ref=vr_ref,
            send_sem=send_sems.at[3], recv_sem=recv_sems.at[3],
            device_id=(left,), device_id_type=pl.DeviceIdType.MESH)

        @pl.when(has_right)
        def _():
            to_right_k.start()
            to_right_v.start()

        @pl.when(has_left)
        def _():
            to_left_k.start()
            to_left_v.start()

        wq_b = wq_ref[...].astype(jnp.bfloat16)
        q = [
            lax.dot(x_ref[b].astype(jnp.bfloat16), wq_b,
                    preferred_element_type=jnp.float32).astype(jnp.bfloat16)
            for b in range(B)
        ]

        q_idx = lax.broadcasted_iota(jnp.int32, (SQ, KF), 0)
        k_idx = lax.broadcasted_iota(jnp.int32, (SQ, KF), 1)
        k_g = my * SKV - W + k_idx
        mask = ((k_idx >= q_idx) & (k_idx <= q_idx + 2 * W)
                & (k_g >= 0) & (k_g < N_DEV * SKV))

        @pl.when(has_left)
        def _():
            to_right_k.wait_recv()
            to_right_v.wait_recv()

        @pl.when(has_right)
        def _():
            to_left_k.wait_recv()
            to_left_v.wait_recv()

        wo_b = wo_ref[...].astype(jnp.bfloat16)
        for b in range(B):
            ctx_pieces = []
            for h in range(HQ):
                kf = jnp.concatenate(
                    [kl_ref[b, :, h, :], k_ref[b, :, h, :], kr_ref[b, :, h, :]],
                    axis=0).astype(jnp.bfloat16)
                vf = jnp.concatenate(
                    [vl_ref[b, :, h, :], v_ref[b, :, h, :], vr_ref[b, :, h, :]],
                    axis=0).astype(jnp.bfloat16)
                qh = q[b][:, h * DH:(h + 1) * DH]
                s = lax.dot_general(qh, kf, (((1,), (1,)), ((), ())),
                                    preferred_element_type=jnp.float32) * 0.125
                s = jnp.where(mask, s, -1e9)
                m = jnp.max(s, axis=1, keepdims=True)
                w = jnp.exp(s - m)
                w = w / jnp.sum(w, axis=1, keepdims=True)
                ctx_pieces.append(
                    lax.dot(w.astype(jnp.bfloat16), vf,
                            preferred_element_type=jnp.float32)
                    .astype(jnp.bfloat16))
            ctx = jnp.concatenate(ctx_pieces, axis=1)
            out_ref[b] = lax.dot(ctx, wo_b,
                                 preferred_element_type=jnp.float32)

        @pl.when(has_right)
        def _():
            to_right_k.wait_send()
            to_right_v.wait_send()

        @pl.when(has_left)
        def _():
            to_left_k.wait_send()
            to_left_v.wait_send()

    return pl.pallas_call(
        body,
        out_shape=jax.ShapeDtypeStruct((B, SQ, DM), jnp.float32),
        in_specs=[pl.BlockSpec(memory_space=pltpu.VMEM)] * 5,
        out_specs=pl.BlockSpec(memory_space=pltpu.VMEM),
        scratch_shapes=[
            pltpu.VMEM((B, W, HQ, DH), jnp.float32),
            pltpu.VMEM((B, W, HQ, DH), jnp.float32),
            pltpu.VMEM((B, W, HQ, DH), jnp.float32),
            pltpu.VMEM((B, W, HQ, DH), jnp.float32),
            pltpu.SemaphoreType.DMA((4,)),
            pltpu.SemaphoreType.DMA((4,)),
        ],
        compiler_params=pltpu.CompilerParams(collective_id=0),
    )(x, Wq, K_ext, V_ext, Wo)


# device time: 40459 ns/iter; 1.2876x vs baseline; 1.2876x over previous
import jax
import jax.numpy as jnp
from jax import lax
from jax.experimental import pallas as pl
from jax.experimental.pallas import tpu as pltpu

N_DEV = 8
B = 2
SQ = 512
SKV = 512
HQ = 8
DH = 64
DM = 768
W = 128
KF = SKV + 2 * W
QB = 128
NQB = SQ // QB
KB = QB + 2 * W


def kernel(x, Wq, K_ext, V_ext, Wo):
    def body(x_ref, wq_ref, k_ref, v_ref, wo_ref, out_ref,
             kfull, vfull, sbkl, sbkr, sbvl, sbvr, ctx_ref,
             send_sems, recv_sems):
        my = lax.axis_index("i")
        has_left = my > 0
        has_right = my < N_DEV - 1
        left = jnp.maximum(my - 1, 0)
        right = jnp.minimum(my + 1, N_DEV - 1)

        sbkl[...] = k_ref[:, pl.ds(0, W)].astype(jnp.bfloat16)
        sbvl[...] = v_ref[:, pl.ds(0, W)].astype(jnp.bfloat16)
        sbkr[...] = k_ref[:, pl.ds(SKV - W, W)].astype(jnp.bfloat16)
        sbvr[...] = v_ref[:, pl.ds(SKV - W, W)].astype(jnp.bfloat16)

        @pl.when(jnp.logical_not(has_left))
        def _():
            z = jnp.zeros((B, W, HQ, DH), jnp.bfloat16)
            kfull[:, pl.ds(0, W)] = z
            vfull[:, pl.ds(0, W)] = z

        @pl.when(jnp.logical_not(has_right))
        def _():
            z = jnp.zeros((B, W, HQ, DH), jnp.bfloat16)
            kfull[:, pl.ds(W + SKV, W)] = z
            vfull[:, pl.ds(W + SKV, W)] = z

        barrier = pltpu.get_barrier_semaphore()

        @pl.when(has_left)
        def _():
            pl.semaphore_signal(barrier, inc=1, device_id=(left,),
                                device_id_type=pl.DeviceIdType.MESH)

        @pl.when(has_right)
        def _():
            pl.semaphore_signal(barrier, inc=1, device_id=(right,),
                                device_id_type=pl.DeviceIdType.MESH)

        @pl.when(has_left)
        def _():
            pl.semaphore_wait(barrier, 1)

        @pl.when(has_right)
        def _():
            pl.semaphore_wait(barrier, 1)

        to_right_k = pltpu.make_async_remote_copy(
            src_ref=sbkr, dst_ref=kfull.at[:, pl.ds(0, W)],
            send_sem=send_sems.at[0], recv_sem=recv_sems.at[0],
            device_id=(right,), device_id_type=pl.DeviceIdType.MESH)
        to_right_v = pltpu.make_async_remote_copy(
            src_ref=sbvr, dst_ref=vfull.at[:, pl.ds(0, W)],
            send_sem=send_sems.at[2], recv_sem=recv_sems.at[2],
            device_id=(right,), device_id_type=pl.DeviceIdType.MESH)
        to_left_k = pltpu.make_async_remote_copy(
            src_ref=sbkl, dst_ref=kfull.at[:, pl.ds(W + SKV, W)],
            send_sem=send_sems.at[1], recv_sem=recv_sems.at[1],
            device_id=(left,), device_id_type=pl.DeviceIdType.MESH)
        to_left_v = pltpu.make_async_remote_copy(
            src_ref=sbvl, dst_ref=vfull.at[:, pl.ds(W + SKV, W)],
            send_sem=send_sems.at[3], recv_sem=recv_sems.at[3],
            device_id=(left,), device_id_type=pl.DeviceIdType.MESH)

        @pl.when(has_right)
        def _():
            to_right_k.start()
            to_right_v.start()

        @pl.when(has_left)
        def _():
            to_left_k.start()
            to_left_v.start()

        kfull[:, pl.ds(W, SKV)] = k_ref[...].astype(jnp.bfloat16)
        vfull[:, pl.ds(W, SKV)] = v_ref[...].astype(jnp.bfloat16)

        wq_b = wq_ref[...].astype(jnp.bfloat16)
        q = [
            (lax.dot(x_ref[b].astype(jnp.bfloat16), wq_b,
                     preferred_element_type=jnp.float32)
             * 0.125).astype(jnp.bfloat16)
            for b in range(B)
        ]

        i0 = lax.broadcasted_iota(jnp.int32, (QB, KB), 0)
        j0 = lax.broadcasted_iota(jnp.int32, (QB, KB), 1)
        window = (j0 >= i0) & (j0 <= i0 + 2 * W)
        biases = []
        for qb in range(NQB):
            k_g = my * SKV - W + qb * QB + j0
            valid = (k_g >= 0) & (k_g < N_DEV * SKV)
            biases.append(jnp.where(window & valid, 0.0, -1e9)
                          .astype(jnp.float32))

        ones_kb = jnp.ones((KB, 1), jnp.bfloat16)

        def attn(b, h, qb):
            qh = q[b][qb * QB:(qb + 1) * QB, h * DH:(h + 1) * DH]
            kf = kfull[b, pl.ds(qb * QB, KB), h, :]
            vf = vfull[b, pl.ds(qb * QB, KB), h, :]
            s = lax.dot_general(qh, kf, (((1,), (1,)), ((), ())),
                                preferred_element_type=jnp.float32)
            w = jnp.exp((s + biases[qb]).astype(jnp.bfloat16))
            ssum = lax.dot(w, ones_kb, preferred_element_type=jnp.float32)
            ctxp = lax.dot(w, vf, preferred_element_type=jnp.float32)
            ctx_ref[b, pl.ds(qb * QB, QB), pl.ds(h * DH, DH)] = (
                ctxp / ssum).astype(jnp.bfloat16)

        for b in range(B):
            for h in range(HQ):
                attn(b, h, 1)
                attn(b, h, 2)

        @pl.when(has_left)
        def _():
            to_right_k.wait_recv()
            to_right_v.wait_recv()

        for b in range(B):
            for h in range(HQ):
                attn(b, h, 0)

        @pl.when(has_right)
        def _():
            to_left_k.wait_recv()
            to_left_v.wait_recv()

        for b in range(B):
            for h in range(HQ):
                attn(b, h, 3)

        wo_b = wo_ref[...].astype(jnp.bfloat16)
        for b in range(B):
            out_ref[b] = lax.dot(ctx_ref[b], wo_b,
                                 preferred_element_type=jnp.float32)

        @pl.when(has_right)
        def _():
            to_right_k.wait_send()
            to_right_v.wait_send()

        @pl.when(has_left)
        def _():
            to_left_k.wait_send()
            to_left_v.wait_send()

    return pl.pallas_call(
        body,
        out_shape=jax.ShapeDtypeStruct((B, SQ, DM), jnp.float32),
        in_specs=[pl.BlockSpec(memory_space=pltpu.VMEM)] * 5,
        out_specs=pl.BlockSpec(memory_space=pltpu.VMEM),
        scratch_shapes=[
            pltpu.VMEM((B, KF, HQ, DH), jnp.bfloat16),
            pltpu.VMEM((B, KF, HQ, DH), jnp.bfloat16),
            pltpu.VMEM((B, W, HQ, DH), jnp.bfloat16),
            pltpu.VMEM((B, W, HQ, DH), jnp.bfloat16),
            pltpu.VMEM((B, W, HQ, DH), jnp.bfloat16),
            pltpu.VMEM((B, W, HQ, DH), jnp.bfloat16),
            pltpu.VMEM((B, SQ, HQ * DH), jnp.bfloat16),
            pltpu.SemaphoreType.DMA((4,)),
            pltpu.SemaphoreType.DMA((4,)),
        ],
        compiler_params=pltpu.CompilerParams(collective_id=0),
    )(x, Wq, K_ext, V_ext, Wo)


# device time: 26910 ns/iter; 1.9359x vs baseline; 1.5035x over previous
import jax
import jax.numpy as jnp
from jax import lax
from jax.experimental import pallas as pl
from jax.experimental.pallas import tpu as pltpu

N_DEV = 8
B = 2
SQ = 512
SKV = 512
HQ = 8
DH = 64
DM = 768
W = 128
KF = SKV + 2 * W
QB = 128
NQB = SQ // QB
KB = QB + 2 * W

COMM = True


def kernel(x, Wq, K_ext, V_ext, Wo):
    def body(x_ref, wq_ref, k_ref, v_ref, wo_ref, out_ref,
             kfull, vfull, sbkl, sbkr, sbvl, sbvr,
             hkl, hkr, hvl, hvr, ctx_ref,
             send_sems, recv_sems):
        scope = jax.named_scope
        my = lax.axis_index("i")
        has_left = my > 0
        has_right = my < N_DEV - 1
        left = jnp.maximum(my - 1, 0)
        right = jnp.minimum(my + 1, N_DEV - 1)

        with scope("stage"):
            sbkl[...] = k_ref[:, :, pl.ds(0, W)].astype(jnp.float8_e4m3fn)
            sbvl[...] = v_ref[:, :, pl.ds(0, W)].astype(jnp.float8_e4m3fn)
            sbkr[...] = k_ref[:, :, pl.ds(SKV - W, W)].astype(jnp.float8_e4m3fn)
            sbvr[...] = v_ref[:, :, pl.ds(SKV - W, W)].astype(jnp.float8_e4m3fn)

            no_left = jnp.logical_not(has_left) if COMM else jnp.bool_(True)
            no_right = jnp.logical_not(has_right) if COMM else jnp.bool_(True)

            @pl.when(no_left)
            def _():
                z = jnp.zeros((B, HQ, W, DH), jnp.bfloat16)
                kfull[:, :, pl.ds(0, W)] = z
                vfull[:, :, pl.ds(0, W)] = z

            @pl.when(no_right)
            def _():
                z = jnp.zeros((B, HQ, W, DH), jnp.bfloat16)
                kfull[:, :, pl.ds(W + SKV, W)] = z
                vfull[:, :, pl.ds(W + SKV, W)] = z

        def _barrier():
            barrier = pltpu.get_barrier_semaphore()

            @pl.when(has_left)
            def _():
                pl.semaphore_signal(barrier, inc=1, device_id=(left,),
                                    device_id_type=pl.DeviceIdType.MESH)

            @pl.when(has_right)
            def _():
                pl.semaphore_signal(barrier, inc=1, device_id=(right,),
                                    device_id_type=pl.DeviceIdType.MESH)

            @pl.when(has_left)
            def _():
                pl.semaphore_wait(barrier, 1)

            @pl.when(has_right)
            def _():
                pl.semaphore_wait(barrier, 1)

        if COMM:
            with scope("barrier"):
                _barrier()

        to_right_k = pltpu.make_async_remote_copy(
            src_ref=sbkr, dst_ref=hkl,
            send_sem=send_sems.at[0], recv_sem=recv_sems.at[0],
            device_id=(right,), device_id_type=pl.DeviceIdType.MESH)
        to_right_v = pltpu.make_async_remote_copy(
            src_ref=sbvr, dst_ref=hvl,
            send_sem=send_sems.at[2], recv_sem=recv_sems.at[2],
            device_id=(right,), device_id_type=pl.DeviceIdType.MESH)
        to_left_k = pltpu.make_async_remote_copy(
            src_ref=sbkl, dst_ref=hkr,
            send_sem=send_sems.at[1], recv_sem=recv_sems.at[1],
            device_id=(left,), device_id_type=pl.DeviceIdType.MESH)
        to_left_v = pltpu.make_async_remote_copy(
            src_ref=sbvl, dst_ref=hvr,
            send_sem=send_sems.at[3], recv_sem=recv_sems.at[3],
            device_id=(left,), device_id_type=pl.DeviceIdType.MESH)

        if COMM:
          with scope("send_start"):
            @pl.when(has_right)
            def _():
                to_right_k.start()
                to_right_v.start()

            @pl.when(has_left)
            def _():
                to_left_k.start()
                to_left_v.start()

        with scope("fill_kv"):
            kfull[:, :, pl.ds(W, SKV)] = k_ref[...].astype(jnp.bfloat16)
            vfull[:, :, pl.ds(W, SKV)] = v_ref[...].astype(jnp.bfloat16)

        with scope("qproj"):
            wq_b = (wq_ref[...] * 0.125).astype(jnp.bfloat16)
            q = [
                lax.dot(x_ref[b].astype(jnp.bfloat16), wq_b,
                        preferred_element_type=jnp.float32)
                .astype(jnp.bfloat16)
                for b in range(B)
            ]

        wo_b = wo_ref[...].astype(jnp.bfloat16)

        i0 = lax.broadcasted_iota(jnp.int32, (QB, KB), 0)
        j0 = lax.broadcasted_iota(jnp.int32, (QB, KB), 1)
        window = (j0 >= i0) & (j0 <= i0 + 2 * W)
        biases = []
        for qb in range(NQB):
            k_g = my * SKV - W + qb * QB + j0
            valid = (k_g >= 0) & (k_g < N_DEV * SKV)
            biases.append(jnp.where(window & valid, 0.0, -1e9)
                          .astype(jnp.bfloat16))

        ones_kb = jnp.ones((KB, 1), jnp.bfloat16)

        def attn(b, h, qb):
            qh = q[b][qb * QB:(qb + 1) * QB, h * DH:(h + 1) * DH]
            kf = kfull[b, h, pl.ds(qb * QB, KB), :]
            vf = vfull[b, h, pl.ds(qb * QB, KB), :]
            s = lax.dot_general(qh, kf, (((1,), (1,)), ((), ())),
                                preferred_element_type=jnp.float32)
            w = jnp.exp(s.astype(jnp.bfloat16) + biases[qb])
            ssum = lax.dot(w, ones_kb, preferred_element_type=jnp.float32)
            ctxp = lax.dot(w, vf, preferred_element_type=jnp.float32)
            ctx_ref[b, pl.ds(qb * QB, QB), pl.ds(h * DH, DH)] = (
                ctxp / ssum).astype(jnp.bfloat16)

        def outproj(b, row0, nrows):
            out_ref[b, pl.ds(row0, nrows)] = lax.dot(
                ctx_ref[b, pl.ds(row0, nrows), :], wo_b,
                preferred_element_type=jnp.float32)

        with scope("attn_mid"):
            for b in range(B):
                for h in range(HQ):
                    attn(b, h, 1)
                    attn(b, h, 2)
            for b in range(B):
                outproj(b, QB, 2 * QB)

        if COMM:
          with scope("wait_left"):
            @pl.when(has_left)
            def _():
                to_right_k.wait_recv()
                to_right_v.wait_recv()
                kfull[:, :, pl.ds(0, W)] = hkl[...].astype(jnp.bfloat16)
                vfull[:, :, pl.ds(0, W)] = hvl[...].astype(jnp.bfloat16)

        with scope("attn_qb0"):
            for b in range(B):
                for h in range(HQ):
                    attn(b, h, 0)
            for b in range(B):
                outproj(b, 0, QB)

        if COMM:
          with scope("wait_right"):
            @pl.when(has_right)
            def _():
                to_left_k.wait_recv()
                to_left_v.wait_recv()
                kfull[:, :, pl.ds(W + SKV, W)] = hkr[...].astype(jnp.bfloat16)
                vfull[:, :, pl.ds(W + SKV, W)] = hvr[...].astype(jnp.bfloat16)

        with scope("attn_qb3"):
            for b in range(B):
                for h in range(HQ):
                    attn(b, h, 3)
            for b in range(B):
                outproj(b, 3 * QB, QB)

        if COMM:
          with scope("drain"):
            @pl.when(has_right)
            def _():
                to_right_k.wait_send()
                to_right_v.wait_send()

            @pl.when(has_left)
            def _():
                to_left_k.wait_send()
                to_left_v.wait_send()

    return pl.pallas_call(
        body,
        out_shape=jax.ShapeDtypeStruct((B, SQ, DM), jnp.float32),
        in_specs=[pl.BlockSpec(memory_space=pltpu.VMEM)] * 5,
        out_specs=pl.BlockSpec(memory_space=pltpu.VMEM),
        scratch_shapes=[
            pltpu.VMEM((B, HQ, KF, DH), jnp.bfloat16),
            pltpu.VMEM((B, HQ, KF, DH), jnp.bfloat16),
            pltpu.VMEM((B, HQ, W, DH), jnp.float8_e4m3fn),
            pltpu.VMEM((B, HQ, W, DH), jnp.float8_e4m3fn),
            pltpu.VMEM((B, HQ, W, DH), jnp.float8_e4m3fn),
            pltpu.VMEM((B, HQ, W, DH), jnp.float8_e4m3fn),
            pltpu.VMEM((B, HQ, W, DH), jnp.float8_e4m3fn),
            pltpu.VMEM((B, HQ, W, DH), jnp.float8_e4m3fn),
            pltpu.VMEM((B, HQ, W, DH), jnp.float8_e4m3fn),
            pltpu.VMEM((B, HQ, W, DH), jnp.float8_e4m3fn),
            pltpu.VMEM((B, SQ, HQ * DH), jnp.bfloat16),
            pltpu.SemaphoreType.DMA((4,)),
            pltpu.SemaphoreType.DMA((4,)),
        ],
        compiler_params=(pltpu.CompilerParams(collective_id=0) if COMM
                         else pltpu.CompilerParams()),
    )(x, Wq, jnp.transpose(K_ext, (0, 2, 1, 3)),
      jnp.transpose(V_ext, (0, 2, 1, 3)), Wo)
